# baseline (speedup 1.0000x reference)
import functools

import jax
import jax.numpy as jnp
from jax import lax
from jax.experimental import pallas as pl
from jax.experimental.pallas import tpu as pltpu
from jax.experimental.pallas import tpu_sc as plsc

NU = 50000
NUP = 50048
D = 64
DH = 32
E = 800000
CH = 128
ECH = E // CH
ECHP = 6256
TCH = 2 * E // CH
TCHP = 12512
NW = 32
NS = 16
HP = 50048
HSL = HP // NS
ASL = NUP // NS
NB1 = 400
NB3 = 208
BM = 2000

_mesh = plsc.VectorSubcoreMesh(core_axis_name="c", subcore_axis_name="s")


def _chunk_range(wid, total):
    c0 = wid * total // NW
    c1 = (wid + 1) * total // NW
    r0 = pl.multiple_of((c0 // 8) * 8, 8)
    return c0, c1, r0


@functools.partial(
    pl.kernel,
    compiler_params=pltpu.CompilerParams(use_tc_tiling_on_sc=False),
    out_type=(jax.ShapeDtypeStruct((HP, 8), jnp.float32),
              jax.ShapeDtypeStruct((HP, 8), jnp.float32)),
    mesh=_mesh,
    scratch_types=[pltpu.VMEM_SHARED((HP, 8), jnp.float32),
                   pltpu.VMEM((NB1, CH), jnp.int32),
                   pltpu.VMEM((CH, 8), jnp.float32)],
)
def _hist(idx_hbm, ones_hbm, zeros_hbm, cnt_a, cnt_b, acc_sh, idx_v, ones_v):
    c = lax.axis_index("c")
    s = lax.axis_index("s")
    wid = c * NS + s
    pltpu.sync_copy(zeros_hbm, acc_sh.at[pl.ds(s * HSL, HSL)])
    pltpu.sync_copy(ones_hbm, ones_v)
    c0, c1, r0 = _chunk_range(wid, TCH)
    off = c0 - r0
    pltpu.sync_copy(idx_hbm.at[pl.ds(r0, NB1)], idx_v)
    plsc.subcore_barrier()

    def body(j, carry):
        pltpu.sync_copy(ones_v, acc_sh.at[idx_v.at[j]], add=True)
        return carry

    lax.fori_loop(off, off + (c1 - c0), body, 0)
    plsc.subcore_barrier()

    @pl.when(c == 0)
    def _():
        pltpu.sync_copy(acc_sh.at[pl.ds(s * HSL, HSL)],
                        cnt_a.at[pl.ds(s * HSL, HSL)])

    @pl.when(c == 1)
    def _():
        pltpu.sync_copy(acc_sh.at[pl.ds(s * HSL, HSL)],
                        cnt_b.at[pl.ds(s * HSL, HSL)])


@functools.partial(
    pl.kernel,
    compiler_params=pltpu.CompilerParams(use_tc_tiling_on_sc=False),
    out_type=(jax.ShapeDtypeStruct((NUP, DH), jnp.float32),
              jax.ShapeDtypeStruct((NUP, DH), jnp.float32)),
    mesh=_mesh,
    scratch_types=[pltpu.VMEM_SHARED((NUP, DH), jnp.float32),
                   pltpu.VMEM((8, CH), jnp.int32),
                   pltpu.VMEM((8, CH), jnp.int32),
                   pltpu.VMEM((CH, DH), jnp.float32),
                   pltpu.VMEM((CH, DH), jnp.float32)],
)
def _spmm(hda, hdb, iu_hbm, ip_hbm, acc_a, acc_b,
          acc_sh, iu_v, ip_v, ru_v, rp_v):
    c = lax.axis_index("c")
    s = lax.axis_index("s")
    c0 = s * ECH // NS
    c1 = (s + 1) * ECH // NS
    g0 = c0 // 8
    g1 = (c1 + 7) // 8

    def run(hd_half, acc_out):
        pltpu.sync_copy(hd_half.at[pl.ds(s * ASL, ASL)],
                        acc_sh.at[pl.ds(s * ASL, ASL)])
        plsc.subcore_barrier()

        def grp(m, carry):
            base = pl.multiple_of(m * 8, 8)
            pltpu.sync_copy(iu_hbm.at[pl.ds(base, 8)], iu_v)
            pltpu.sync_copy(ip_hbm.at[pl.ds(base, 8)], ip_v)

            def body(j, carry2):
                gidx = base + j

                @pl.when(jnp.logical_and(gidx >= c0, gidx < c1))
                def _():
                    pltpu.sync_copy(hd_half.at[iu_v.at[j]], ru_v)
                    pltpu.sync_copy(hd_half.at[ip_v.at[j]], rp_v)
                    pltpu.sync_copy(ru_v, acc_sh.at[ip_v.at[j]], add=True)
                    pltpu.sync_copy(rp_v, acc_sh.at[iu_v.at[j]], add=True)

                return carry2

            lax.fori_loop(0, 8, body, 0)
            return carry

        lax.fori_loop(g0, g1, grp, 0)
        plsc.subcore_barrier()
        pltpu.sync_copy(acc_sh.at[pl.ds(s * ASL, ASL)],
                        acc_out.at[pl.ds(s * ASL, ASL)])

    @pl.when(c == 0)
    def _():
        run(hda, acc_a)

    @pl.when(c == 1)
    def _():
        run(hdb, acc_b)


@functools.partial(
    pl.kernel,
    compiler_params=pltpu.CompilerParams(use_tc_tiling_on_sc=False,
                                         needs_layout_passes=False),
    out_type=jax.ShapeDtypeStruct((E,), jnp.float32),
    mesh=_mesh,
    scratch_types=[pltpu.VMEM((NB3, CH), jnp.int32),
                   pltpu.VMEM((NB3, CH), jnp.int32),
                   pltpu.VMEM((CH, D), jnp.float32),
                   pltpu.VMEM((CH, D), jnp.float32),
                   pltpu.VMEM((80,), jnp.float32),
                   pltpu.VMEM((CH,), jnp.float32)],
)
def _edge_score(u2_hbm, p2_hbm, iu_hbm, ip_hbm, wb_hbm, out,
                iu_v, ip_v, ru_v, rp_v, wb_v, os_v):
    c = lax.axis_index("c")
    s = lax.axis_index("s")
    wid = c * NS + s
    c0, c1, r0 = _chunk_range(wid, ECH)
    off = c0 - r0
    pltpu.sync_copy(iu_hbm.at[pl.ds(r0, NB3)], iu_v)
    pltpu.sync_copy(ip_hbm.at[pl.ds(r0, NB3)], ip_v)
    pltpu.sync_copy(wb_hbm, wb_v)
    w = [wb_v[pl.ds(16 * k, 16)] for k in range(4)]
    bias = wb_v[pl.ds(64, 16)]
    lanes = lax.iota(jnp.int32, 16)

    def chunk(j, carry):
        pltpu.sync_copy(u2_hbm.at[iu_v.at[j]], ru_v)
        pltpu.sync_copy(p2_hbm.at[ip_v.at[j]], rp_v)

        def strip(g, carry2):
            sv = jnp.zeros((16,), jnp.float32)
            for i in range(16):
                e = g * 16 + i
                acc = jnp.zeros((16,), jnp.float32)
                for k in range(4):
                    u = ru_v[e, pl.ds(16 * k, 16)]
                    p = rp_v[e, pl.ds(16 * k, 16)]
                    acc = acc + jnp.maximum(u + p, 0.0) * w[k]
                sv = jnp.where(lanes == i, jnp.sum(acc), sv)
            os_v[pl.ds(g * 16, 16)] = 5.0 / (1.0 + jnp.exp(-(sv + bias)))
            return carry2

        lax.fori_loop(0, CH // 16, strip, 0)
        pltpu.sync_copy(os_v, out.at[pl.ds((r0 + j) * CH, CH)])
        return carry

    lax.fori_loop(off, off + (c1 - c0), chunk, 0)


def _row(i):
    return (i, 0)


def _full(i):
    return (0, 0)


def _dis_of(ca_r, cb_r):
    return lax.rsqrt(ca_r[:, :1] + cb_r[:, :1] + 1.0)


def _t1_body(uf_r, pf_r, uew_r, pew_r, uftW_r, uftb_r, pftW_r, pftb_r,
             c1W_r, ca_r, cb_r, hda_r, hdb_r, h1p_r):
    dis = _dis_of(ca_r, cb_r)
    ux = jnp.dot(uf_r[:], uftW_r[:], preferred_element_type=jnp.float32)
    ux = ux + uftb_r[:] + uew_r[:]
    h1u = jnp.dot(ux, c1W_r[:], preferred_element_type=jnp.float32)
    hd = h1u * dis
    hda_r[:] = hd[:, :DH]
    hdb_r[:] = hd[:, DH:]
    px = jnp.dot(pf_r[:], pftW_r[:], preferred_element_type=jnp.float32)
    px = px + pftb_r[:] + pew_r[:]
    h1p_r[:] = jnp.dot(px, c1W_r[:], preferred_element_type=jnp.float32)


def _t2_body(aa_r, ab_r, ca_r, cb_r, h1p_r, c1b_r, c2W_r,
             hda_r, hdb_r, h2p_r):
    dis = _dis_of(ca_r, cb_r)
    acc = jnp.concatenate([aa_r[:], ab_r[:]], axis=1)
    u1 = jnp.maximum(acc * dis + c1b_r[:], 0.0)
    h2u = jnp.dot(u1, c2W_r[:], preferred_element_type=jnp.float32)
    hd2 = h2u * dis
    hda_r[:] = hd2[:, :DH]
    hdb_r[:] = hd2[:, DH:]
    p1 = jnp.maximum(h1p_r[:] + c1b_r[:], 0.0)
    h2p_r[:] = jnp.dot(p1, c2W_r[:], preferred_element_type=jnp.float32)


def _t3_body(aa_r, ab_r, ca_r, cb_r, h2p_r, c2b_r, pWu_r, pWp_r, p1b_r,
             u2_r, p2_r):
    dis = _dis_of(ca_r, cb_r)
    u2 = jnp.concatenate([aa_r[:], ab_r[:]], axis=1) * dis + c2b_r[:]
    u2_r[:] = jnp.dot(u2, pWu_r[:], preferred_element_type=jnp.float32) + p1b_r[:]
    p2 = h2p_r[:] + c2b_r[:]
    p2_r[:] = jnp.dot(p2, pWp_r[:], preferred_element_type=jnp.float32)


def _tc_call(body, in_specs, out_specs, out_shape, args):
    return pl.pallas_call(
        body,
        grid=(NU // BM,),
        in_specs=in_specs,
        out_specs=out_specs,
        out_shape=out_shape,
    )(*args)


def kernel(edge_index, user_features, product_features, ue_w, pe_w, uft_W,
           uft_b, pft_W, pft_b, c1_W, c1_b, c2_W, c2_b, p1_W, p1_b, p2_W,
           p2_b):
    f32 = jnp.float32
    i32 = jnp.int32
    ei = edge_index.astype(i32)
    pad_e = jnp.zeros((ECHP - ECH, CH), i32)
    iu2d = jnp.concatenate([ei[0].reshape(ECH, CH), pad_e])
    ip2d = jnp.concatenate([ei[1].reshape(ECH, CH), pad_e])
    idx_flat = jnp.concatenate(
        [ei.reshape(TCH, CH), jnp.zeros((TCHP - TCH, CH), i32)])
    ones8 = jnp.ones((CH, 8), f32)
    zeros8 = jnp.zeros((HSL, 8), f32)

    cnt_a, cnt_b = _hist(idx_flat, ones8, zeros8)

    mrow = lambda shape: pl.BlockSpec(shape, _row)
    mfull = lambda shape: pl.BlockSpec(shape, _full)
    sd = jax.ShapeDtypeStruct

    hd1a, hd1b, h1p = _tc_call(
        _t1_body,
        [mrow((BM, 128)), mrow((BM, 128)), mrow((BM, D)), mrow((BM, D)),
         mfull((128, D)), mfull((1, D)), mfull((128, D)), mfull((1, D)),
         mfull((D, D)), mrow((BM, 8)), mrow((BM, 8))],
        [mrow((BM, DH)), mrow((BM, DH)), mrow((BM, D))],
        [sd((NUP, DH), f32), sd((NUP, DH), f32), sd((NU, D), f32)],
        (user_features, product_features, ue_w, pe_w, uft_W,
         uft_b.reshape(1, D), pft_W, pft_b.reshape(1, D), c1_W, cnt_a,
         cnt_b))

    acc1a, acc1b = _spmm(hd1a, hd1b, iu2d, ip2d)

    hd2a, hd2b, h2p = _tc_call(
        _t2_body,
        [mrow((BM, DH)), mrow((BM, DH)), mrow((BM, 8)), mrow((BM, 8)),
         mrow((BM, D)), mfull((1, D)), mfull((D, D))],
        [mrow((BM, DH)), mrow((BM, DH)), mrow((BM, D))],
        [sd((NUP, DH), f32), sd((NUP, DH), f32), sd((NU, D), f32)],
        (acc1a, acc1b, cnt_a, cnt_b, h1p, c1_b.reshape(1, D), c2_W))

    acc2a, acc2b = _spmm(hd2a, hd2b, iu2d, ip2d)

    u2, p2 = _tc_call(
        _t3_body,
        [mrow((BM, DH)), mrow((BM, DH)), mrow((BM, 8)), mrow((BM, 8)),
         mrow((BM, D)), mfull((1, D)), mfull((D, D)), mfull((D, D)),
         mfull((1, D))],
        [mrow((BM, D)), mrow((BM, D))],
        [sd((NU, D), f32), sd((NU, D), f32)],
        (acc2a, acc2b, cnt_a, cnt_b, h2p, c2_b.reshape(1, D), p1_W[:D],
         p1_W[D:], p1_b.reshape(1, D)))

    wb = jnp.concatenate([p2_W[:, 0], jnp.full((16,), p2_b[0], f32)])
    return _edge_score(u2, p2, iu2d, ip2d, wb)


# speedup vs baseline: 1.4020x; 1.4020x over previous
import functools

import jax
import jax.numpy as jnp
from jax import lax
from jax.experimental import pallas as pl
from jax.experimental.pallas import tpu as pltpu
from jax.experimental.pallas import tpu_sc as plsc

NU = 50000
NUP = 50048
D = 64
DH = 32
E = 800000
CH = 128
ECH = E // CH
ECHP = 6264
TCH = 2 * E // CH
TCHP = 12512
NW = 32
NS = 16
HP = 50048
HSL = HP // NS
ASL = NUP // NS
NB1 = 400
NB3 = 208
BM = 2000

_mesh = plsc.VectorSubcoreMesh(core_axis_name="c", subcore_axis_name="s")


def _chunk_range(wid, total):
    c0 = wid * total // NW
    c1 = (wid + 1) * total // NW
    r0 = pl.multiple_of((c0 // 8) * 8, 8)
    return c0, c1, r0


@functools.partial(
    pl.kernel,
    compiler_params=pltpu.CompilerParams(use_tc_tiling_on_sc=False),
    out_type=(jax.ShapeDtypeStruct((HP, 8), jnp.float32),
              jax.ShapeDtypeStruct((HP, 8), jnp.float32)),
    mesh=_mesh,
    scratch_types=[pltpu.VMEM_SHARED((HP, 8), jnp.float32),
                   pltpu.VMEM((NB1, CH), jnp.int32),
                   pltpu.VMEM((CH, 8), jnp.float32)],
)
def _hist(idx_hbm, ones_hbm, zeros_hbm, cnt_a, cnt_b, acc_sh, idx_v, ones_v):
    c = lax.axis_index("c")
    s = lax.axis_index("s")
    wid = c * NS + s
    pltpu.sync_copy(zeros_hbm, acc_sh.at[pl.ds(s * HSL, HSL)])
    pltpu.sync_copy(ones_hbm, ones_v)
    c0, c1, r0 = _chunk_range(wid, TCH)
    off = c0 - r0
    pltpu.sync_copy(idx_hbm.at[pl.ds(r0, NB1)], idx_v)
    plsc.subcore_barrier()

    def body(j, carry):
        pltpu.sync_copy(ones_v, acc_sh.at[idx_v.at[j]], add=True)
        return carry

    lax.fori_loop(off, off + (c1 - c0), body, 0)
    plsc.subcore_barrier()

    @pl.when(c == 0)
    def _():
        pltpu.sync_copy(acc_sh.at[pl.ds(s * HSL, HSL)],
                        cnt_a.at[pl.ds(s * HSL, HSL)])

    @pl.when(c == 1)
    def _():
        pltpu.sync_copy(acc_sh.at[pl.ds(s * HSL, HSL)],
                        cnt_b.at[pl.ds(s * HSL, HSL)])


@functools.partial(
    pl.kernel,
    compiler_params=pltpu.CompilerParams(use_tc_tiling_on_sc=False),
    out_type=(jax.ShapeDtypeStruct((NUP, DH), jnp.float32),
              jax.ShapeDtypeStruct((NUP, DH), jnp.float32)),
    mesh=_mesh,
    scratch_types=[pltpu.VMEM_SHARED((NUP, DH), jnp.float32),
                   pltpu.VMEM((16, CH), jnp.int32),
                   pltpu.VMEM((16, CH), jnp.int32),
                   pltpu.VMEM((CH, DH), jnp.float32),
                   pltpu.VMEM((CH, DH), jnp.float32),
                   pltpu.VMEM((CH, DH), jnp.float32),
                   pltpu.VMEM((CH, DH), jnp.float32),
                   pltpu.SemaphoreType.DMA,
                   pltpu.SemaphoreType.DMA,
                   pltpu.SemaphoreType.DMA,
                   pltpu.SemaphoreType.DMA],
)
def _spmm(hda, hdb, iu_hbm, ip_hbm, acc_a, acc_b,
          acc_sh, iu_v, ip_v, ru0, rp0, ru1, rp1, gs0, gs1, ss0, ss1):
    c = lax.axis_index("c")
    s = lax.axis_index("s")
    c0 = s * ECH // NS
    c1 = (s + 1) * ECH // NS
    g0 = c0 // 8
    g1 = (c1 + 7) // 8
    npair = (g1 - g0 + 1) // 2
    bufs = ((ru0, rp0, gs0, ss0), (ru1, rp1, gs1, ss1))

    def run(hd_half, acc_out):
        pltpu.sync_copy(hd_half.at[pl.ds(s * ASL, ASL)],
                        acc_sh.at[pl.ds(s * ASL, ASL)])
        plsc.subcore_barrier()

        def active(j):
            return jnp.logical_and(j >= c0, j < c1)

        def g_start(k, b):
            ru, rp, gs, _ = bufs[b]
            pltpu.async_copy(hd_half.at[iu_v.at[k]], ru, gs)
            pltpu.async_copy(hd_half.at[ip_v.at[k]], rp, gs)

        def g_wait(k, b):
            ru, rp, gs, _ = bufs[b]
            pltpu.make_async_copy(hd_half.at[iu_v.at[k]], ru, gs).wait()
            pltpu.make_async_copy(hd_half.at[ip_v.at[k]], rp, gs).wait()

        def s_start(k, b):
            ru, rp, _, ss = bufs[b]
            pltpu.async_copy(ru, acc_sh.at[ip_v.at[k]], ss)
            pltpu.async_copy(rp, acc_sh.at[iu_v.at[k]], ss)

        def s_wait(k, b):
            ru, rp, _, ss = bufs[b]
            pltpu.make_async_copy(ru, acc_sh.at[ip_v.at[k]], ss).wait()
            pltpu.make_async_copy(rp, acc_sh.at[iu_v.at[k]], ss).wait()

        def window(t, carry):
            base = pl.multiple_of((g0 + 2 * t) * 8, 8)
            pltpu.sync_copy(iu_hbm.at[pl.ds(base, 16)], iu_v)
            pltpu.sync_copy(ip_hbm.at[pl.ds(base, 16)], ip_v)

            @pl.when(active(base))
            def _():
                g_start(0, 0)

            for k in range(16):
                b = k % 2
                jk = base + k

                @pl.when(active(jk))
                def _(k=k, b=b, jk=jk):
                    g_wait(k, b)
                    s_start(k, b)

                if k >= 1:

                    @pl.when(active(jk - 1))
                    def _(k=k, b=b):
                        s_wait(k - 1, 1 - b)

                if k + 1 < 16:

                    @pl.when(active(jk + 1))
                    def _(k=k, b=b):
                        g_start(k + 1, 1 - b)

            @pl.when(active(base + 15))
            def _():
                s_wait(15, 1)

            return carry

        lax.fori_loop(0, npair, window, 0)
        plsc.subcore_barrier()
        pltpu.sync_copy(acc_sh.at[pl.ds(s * ASL, ASL)],
                        acc_out.at[pl.ds(s * ASL, ASL)])

    @pl.when(c == 0)
    def _():
        run(hda, acc_a)

    @pl.when(c == 1)
    def _():
        run(hdb, acc_b)


@functools.partial(
    pl.kernel,
    compiler_params=pltpu.CompilerParams(use_tc_tiling_on_sc=False,
                                         needs_layout_passes=False),
    out_type=jax.ShapeDtypeStruct((E,), jnp.float32),
    mesh=_mesh,
    scratch_types=[pltpu.VMEM((NB3, CH), jnp.int32),
                   pltpu.VMEM((NB3, CH), jnp.int32),
                   pltpu.VMEM((CH, D), jnp.float32),
                   pltpu.VMEM((CH, D), jnp.float32),
                   pltpu.VMEM((80,), jnp.float32),
                   pltpu.VMEM((CH,), jnp.float32)],
)
def _edge_score(u2_hbm, p2_hbm, iu_hbm, ip_hbm, wb_hbm, out,
                iu_v, ip_v, ru_v, rp_v, wb_v, os_v):
    c = lax.axis_index("c")
    s = lax.axis_index("s")
    wid = c * NS + s
    c0, c1, r0 = _chunk_range(wid, ECH)
    off = c0 - r0
    pltpu.sync_copy(iu_hbm.at[pl.ds(r0, NB3)], iu_v)
    pltpu.sync_copy(ip_hbm.at[pl.ds(r0, NB3)], ip_v)
    pltpu.sync_copy(wb_hbm, wb_v)
    w = [wb_v[pl.ds(16 * k, 16)] for k in range(4)]
    bias = wb_v[pl.ds(64, 16)]
    lanes = lax.iota(jnp.int32, 16)

    def chunk(j, carry):
        pltpu.sync_copy(u2_hbm.at[iu_v.at[j]], ru_v)
        pltpu.sync_copy(p2_hbm.at[ip_v.at[j]], rp_v)

        def strip(g, carry2):
            sv = jnp.zeros((16,), jnp.float32)
            for i in range(16):
                e = g * 16 + i
                acc = jnp.zeros((16,), jnp.float32)
                for k in range(4):
                    u = ru_v[e, pl.ds(16 * k, 16)]
                    p = rp_v[e, pl.ds(16 * k, 16)]
                    acc = acc + jnp.maximum(u + p, 0.0) * w[k]
                sv = jnp.where(lanes == i, jnp.sum(acc), sv)
            os_v[pl.ds(g * 16, 16)] = 5.0 / (1.0 + jnp.exp(-(sv + bias)))
            return carry2

        lax.fori_loop(0, CH // 16, strip, 0)
        pltpu.sync_copy(os_v, out.at[pl.ds((r0 + j) * CH, CH)])
        return carry

    lax.fori_loop(off, off + (c1 - c0), chunk, 0)


def _row(i):
    return (i, 0)


def _full(i):
    return (0, 0)


def _dis_of(ca_r, cb_r):
    return lax.rsqrt(ca_r[:, :1] + cb_r[:, :1] + 1.0)


def _t1_body(uf_r, pf_r, uew_r, pew_r, uftW_r, uftb_r, pftW_r, pftb_r,
             c1W_r, ca_r, cb_r, hda_r, hdb_r, h1p_r):
    dis = _dis_of(ca_r, cb_r)
    ux = jnp.dot(uf_r[:], uftW_r[:], preferred_element_type=jnp.float32)
    ux = ux + uftb_r[:] + uew_r[:]
    h1u = jnp.dot(ux, c1W_r[:], preferred_element_type=jnp.float32)
    hd = h1u * dis
    hda_r[:] = hd[:, :DH]
    hdb_r[:] = hd[:, DH:]
    px = jnp.dot(pf_r[:], pftW_r[:], preferred_element_type=jnp.float32)
    px = px + pftb_r[:] + pew_r[:]
    h1p_r[:] = jnp.dot(px, c1W_r[:], preferred_element_type=jnp.float32)


def _t2_body(aa_r, ab_r, ca_r, cb_r, h1p_r, c1b_r, c2W_r,
             hda_r, hdb_r, h2p_r):
    dis = _dis_of(ca_r, cb_r)
    acc = jnp.concatenate([aa_r[:], ab_r[:]], axis=1)
    u1 = jnp.maximum(acc * dis + c1b_r[:], 0.0)
    h2u = jnp.dot(u1, c2W_r[:], preferred_element_type=jnp.float32)
    hd2 = h2u * dis
    hda_r[:] = hd2[:, :DH]
    hdb_r[:] = hd2[:, DH:]
    p1 = jnp.maximum(h1p_r[:] + c1b_r[:], 0.0)
    h2p_r[:] = jnp.dot(p1, c2W_r[:], preferred_element_type=jnp.float32)


def _t3_body(aa_r, ab_r, ca_r, cb_r, h2p_r, c2b_r, pWu_r, pWp_r, p1b_r,
             u2_r, p2_r):
    dis = _dis_of(ca_r, cb_r)
    u2 = jnp.concatenate([aa_r[:], ab_r[:]], axis=1) * dis + c2b_r[:]
    u2_r[:] = jnp.dot(u2, pWu_r[:], preferred_element_type=jnp.float32) + p1b_r[:]
    p2 = h2p_r[:] + c2b_r[:]
    p2_r[:] = jnp.dot(p2, pWp_r[:], preferred_element_type=jnp.float32)


def _tc_call(body, in_specs, out_specs, out_shape, args):
    return pl.pallas_call(
        body,
        grid=(NU // BM,),
        in_specs=in_specs,
        out_specs=out_specs,
        out_shape=out_shape,
    )(*args)


def kernel(edge_index, user_features, product_features, ue_w, pe_w, uft_W,
           uft_b, pft_W, pft_b, c1_W, c1_b, c2_W, c2_b, p1_W, p1_b, p2_W,
           p2_b):
    f32 = jnp.float32
    i32 = jnp.int32
    ei = edge_index.astype(i32)
    pad_e = jnp.zeros((ECHP - ECH, CH), i32)
    iu2d = jnp.concatenate([ei[0].reshape(ECH, CH), pad_e])
    ip2d = jnp.concatenate([ei[1].reshape(ECH, CH), pad_e])
    idx_flat = jnp.concatenate(
        [ei.reshape(TCH, CH), jnp.zeros((TCHP - TCH, CH), i32)])
    ones8 = jnp.ones((CH, 8), f32)
    zeros8 = jnp.zeros((HSL, 8), f32)

    cnt_a, cnt_b = _hist(idx_flat, ones8, zeros8)

    mrow = lambda shape: pl.BlockSpec(shape, _row)
    mfull = lambda shape: pl.BlockSpec(shape, _full)
    sd = jax.ShapeDtypeStruct

    hd1a, hd1b, h1p = _tc_call(
        _t1_body,
        [mrow((BM, 128)), mrow((BM, 128)), mrow((BM, D)), mrow((BM, D)),
         mfull((128, D)), mfull((1, D)), mfull((128, D)), mfull((1, D)),
         mfull((D, D)), mrow((BM, 8)), mrow((BM, 8))],
        [mrow((BM, DH)), mrow((BM, DH)), mrow((BM, D))],
        [sd((NUP, DH), f32), sd((NUP, DH), f32), sd((NU, D), f32)],
        (user_features, product_features, ue_w, pe_w, uft_W,
         uft_b.reshape(1, D), pft_W, pft_b.reshape(1, D), c1_W, cnt_a,
         cnt_b))

    acc1a, acc1b = _spmm(hd1a, hd1b, iu2d, ip2d)

    hd2a, hd2b, h2p = _tc_call(
        _t2_body,
        [mrow((BM, DH)), mrow((BM, DH)), mrow((BM, 8)), mrow((BM, 8)),
         mrow((BM, D)), mfull((1, D)), mfull((D, D))],
        [mrow((BM, DH)), mrow((BM, DH)), mrow((BM, D))],
        [sd((NUP, DH), f32), sd((NUP, DH), f32), sd((NU, D), f32)],
        (acc1a, acc1b, cnt_a, cnt_b, h1p, c1_b.reshape(1, D), c2_W))

    acc2a, acc2b = _spmm(hd2a, hd2b, iu2d, ip2d)

    u2, p2 = _tc_call(
        _t3_body,
        [mrow((BM, DH)), mrow((BM, DH)), mrow((BM, 8)), mrow((BM, 8)),
         mrow((BM, D)), mfull((1, D)), mfull((D, D)), mfull((D, D)),
         mfull((1, D))],
        [mrow((BM, D)), mrow((BM, D))],
        [sd((NU, D), f32), sd((NU, D), f32)],
        (acc2a, acc2b, cnt_a, cnt_b, h2p, c2_b.reshape(1, D), p1_W[:D],
         p1_W[D:], p1_b.reshape(1, D)))

    wb = jnp.concatenate([p2_W[:, 0], jnp.full((16,), p2_b[0], f32)])
    return _edge_score(u2, p2, iu2d, ip2d, wb)


# speedup vs baseline: 1.6587x; 1.1831x over previous
import functools

import jax
import jax.numpy as jnp
from jax import lax
from jax.experimental import pallas as pl
from jax.experimental.pallas import tpu as pltpu
from jax.experimental.pallas import tpu_sc as plsc

NU = 50000
NUP = 50048
D = 64
DH = 32
E = 800000
CH = 128
ECH = E // CH
ECHP = 6264
TCH = 2 * E // CH
TCHP = 12512
NW = 32
NS = 16
HP = 50048
HSL = HP // NS
ASL = NUP // NS
NB1 = 400
NB3 = 208
BM = 2000

_mesh = plsc.VectorSubcoreMesh(core_axis_name="c", subcore_axis_name="s")


def _chunk_range(wid, total):
    c0 = wid * total // NW
    c1 = (wid + 1) * total // NW
    r0 = pl.multiple_of((c0 // 8) * 8, 8)
    return c0, c1, r0


@functools.partial(
    pl.kernel,
    compiler_params=pltpu.CompilerParams(use_tc_tiling_on_sc=False),
    out_type=(jax.ShapeDtypeStruct((HP, 8), jnp.float32),
              jax.ShapeDtypeStruct((HP, 8), jnp.float32)),
    mesh=_mesh,
    scratch_types=[pltpu.VMEM_SHARED((HP, 8), jnp.float32),
                   pltpu.VMEM((NB1, CH), jnp.int32),
                   pltpu.VMEM((CH, 8), jnp.float32)],
)
def _hist(idx_hbm, ones_hbm, zeros_hbm, cnt_a, cnt_b, acc_sh, idx_v, ones_v):
    c = lax.axis_index("c")
    s = lax.axis_index("s")
    wid = c * NS + s
    pltpu.sync_copy(zeros_hbm, acc_sh.at[pl.ds(s * HSL, HSL)])
    pltpu.sync_copy(ones_hbm, ones_v)
    c0, c1, r0 = _chunk_range(wid, TCH)
    off = c0 - r0
    pltpu.sync_copy(idx_hbm.at[pl.ds(r0, NB1)], idx_v)
    plsc.subcore_barrier()

    def body(j, carry):
        pltpu.sync_copy(ones_v, acc_sh.at[idx_v.at[j]], add=True)
        return carry

    lax.fori_loop(off, off + (c1 - c0), body, 0)
    plsc.subcore_barrier()

    @pl.when(c == 0)
    def _():
        pltpu.sync_copy(acc_sh.at[pl.ds(s * HSL, HSL)],
                        cnt_a.at[pl.ds(s * HSL, HSL)])

    @pl.when(c == 1)
    def _():
        pltpu.sync_copy(acc_sh.at[pl.ds(s * HSL, HSL)],
                        cnt_b.at[pl.ds(s * HSL, HSL)])


@functools.partial(
    pl.kernel,
    compiler_params=pltpu.CompilerParams(use_tc_tiling_on_sc=False),
    out_type=(jax.ShapeDtypeStruct((NUP, DH), jnp.float32),
              jax.ShapeDtypeStruct((NUP, DH), jnp.float32)),
    mesh=_mesh,
    scratch_types=[pltpu.VMEM_SHARED((NUP, DH), jnp.float32),
                   pltpu.VMEM((16, CH), jnp.int32),
                   pltpu.VMEM((16, CH), jnp.int32),
                   pltpu.VMEM((CH, DH), jnp.float32),
                   pltpu.VMEM((CH, DH), jnp.float32),
                   pltpu.VMEM((CH, DH), jnp.float32),
                   pltpu.VMEM((CH, DH), jnp.float32),
                   pltpu.SemaphoreType.DMA,
                   pltpu.SemaphoreType.DMA,
                   pltpu.SemaphoreType.DMA,
                   pltpu.SemaphoreType.DMA],
)
def _spmm(hda, hdb, iu_hbm, ip_hbm, acc_a, acc_b,
          acc_sh, iu_v, ip_v, ru0, rp0, ru1, rp1, gs0, gs1, ss0, ss1):
    c = lax.axis_index("c")
    s = lax.axis_index("s")
    c0 = s * ECH // NS
    c1 = (s + 1) * ECH // NS
    g0 = c0 // 8
    g1 = (c1 + 7) // 8
    npair = (g1 - g0 + 1) // 2
    bufs = ((ru0, rp0, gs0, ss0), (ru1, rp1, gs1, ss1))

    def run(hd_half, acc_out):
        pltpu.sync_copy(hd_half.at[pl.ds(s * ASL, ASL)],
                        acc_sh.at[pl.ds(s * ASL, ASL)])
        plsc.subcore_barrier()

        def active(j):
            return jnp.logical_and(j >= c0, j < c1)

        def g_start(k, b):
            ru, rp, gs, _ = bufs[b]
            pltpu.async_copy(hd_half.at[iu_v.at[k]], ru, gs)
            pltpu.async_copy(hd_half.at[ip_v.at[k]], rp, gs)

        def g_wait(k, b):
            ru, rp, gs, _ = bufs[b]
            pltpu.make_async_copy(hd_half.at[iu_v.at[k]], ru, gs).wait()
            pltpu.make_async_copy(hd_half.at[ip_v.at[k]], rp, gs).wait()

        def s_start(k, b):
            ru, rp, _, ss = bufs[b]
            pltpu.async_copy(ru, acc_sh.at[ip_v.at[k]], ss, add=True)
            pltpu.async_copy(rp, acc_sh.at[iu_v.at[k]], ss, add=True)

        def s_wait(k, b):
            ru, rp, _, ss = bufs[b]
            pltpu.make_async_copy(ru, acc_sh.at[ip_v.at[k]], ss).wait()
            pltpu.make_async_copy(rp, acc_sh.at[iu_v.at[k]], ss).wait()

        def window(t, carry):
            base = pl.multiple_of((g0 + 2 * t) * 8, 8)
            pltpu.sync_copy(iu_hbm.at[pl.ds(base, 16)], iu_v)
            pltpu.sync_copy(ip_hbm.at[pl.ds(base, 16)], ip_v)

            @pl.when(active(base))
            def _():
                g_start(0, 0)

            for k in range(16):
                b = k % 2
                jk = base + k

                @pl.when(active(jk))
                def _(k=k, b=b, jk=jk):
                    g_wait(k, b)
                    s_start(k, b)

                if k >= 1:

                    @pl.when(active(jk - 1))
                    def _(k=k, b=b):
                        s_wait(k - 1, 1 - b)

                if k + 1 < 16:

                    @pl.when(active(jk + 1))
                    def _(k=k, b=b):
                        g_start(k + 1, 1 - b)

            @pl.when(active(base + 15))
            def _():
                s_wait(15, 1)

            return carry

        lax.fori_loop(0, npair, window, 0)
        plsc.subcore_barrier()
        pltpu.sync_copy(acc_sh.at[pl.ds(s * ASL, ASL)],
                        acc_out.at[pl.ds(s * ASL, ASL)])

    @pl.when(c == 0)
    def _():
        run(hda, acc_a)

    @pl.when(c == 1)
    def _():
        run(hdb, acc_b)


@functools.partial(
    pl.kernel,
    compiler_params=pltpu.CompilerParams(use_tc_tiling_on_sc=False,
                                         needs_layout_passes=False),
    out_type=jax.ShapeDtypeStruct((E,), jnp.float32),
    mesh=_mesh,
    scratch_types=[pltpu.VMEM((NB3, CH), jnp.int32),
                   pltpu.VMEM((NB3, CH), jnp.int32),
                   pltpu.VMEM((CH, D), jnp.float32),
                   pltpu.VMEM((CH, D), jnp.float32),
                   pltpu.VMEM((CH, D), jnp.float32),
                   pltpu.VMEM((CH, D), jnp.float32),
                   pltpu.VMEM((CH,), jnp.float32),
                   pltpu.VMEM((CH,), jnp.float32),
                   pltpu.VMEM((80,), jnp.float32),
                   pltpu.SemaphoreType.DMA,
                   pltpu.SemaphoreType.DMA,
                   pltpu.SemaphoreType.DMA,
                   pltpu.SemaphoreType.DMA],
)
def _edge_score(u2_hbm, p2_hbm, iu_hbm, ip_hbm, wb_hbm, out,
                iu_v, ip_v, ru0, rp0, ru1, rp1, os0, os1, wb_v,
                g0s, g1s, o0s, o1s):
    c = lax.axis_index("c")
    s = lax.axis_index("s")
    wid = c * NS + s
    c0, c1, r0 = _chunk_range(wid, ECH)
    pltpu.sync_copy(iu_hbm.at[pl.ds(r0, NB3)], iu_v)
    pltpu.sync_copy(ip_hbm.at[pl.ds(r0, NB3)], ip_v)
    pltpu.sync_copy(wb_hbm, wb_v)
    w = [wb_v[pl.ds(16 * k, 16)] for k in range(4)]
    bias = wb_v[pl.ds(64, 16)]
    lanes = lax.iota(jnp.int32, 16)
    bufs = ((ru0, rp0, os0, g0s, o0s), (ru1, rp1, os1, g1s, o1s))

    def g_start(j, b):
        ru, rp, _, gs, _ = bufs[b]
        pltpu.async_copy(u2_hbm.at[iu_v.at[j - r0]], ru, gs)
        pltpu.async_copy(p2_hbm.at[ip_v.at[j - r0]], rp, gs)

    def g_wait(j, b):
        ru, rp, _, gs, _ = bufs[b]
        pltpu.make_async_copy(u2_hbm.at[iu_v.at[j - r0]], ru, gs).wait()
        pltpu.make_async_copy(p2_hbm.at[ip_v.at[j - r0]], rp, gs).wait()

    def o_start(j, b):
        _, _, os_v, _, osem = bufs[b]
        pltpu.async_copy(os_v, out.at[pl.ds(j * CH, CH)], osem)

    def o_wait(j, b):
        _, _, os_v, _, osem = bufs[b]
        pltpu.make_async_copy(os_v, out.at[pl.ds(j * CH, CH)], osem).wait()

    def compute(j, b):
        ru, rp, os_v, _, _ = bufs[b]

        def strip(g, carry2):
            sv = jnp.zeros((16,), jnp.float32)
            for i in range(16):
                e = g * 16 + i
                acc = jnp.zeros((16,), jnp.float32)
                for k in range(4):
                    u = ru[e, pl.ds(16 * k, 16)]
                    p = rp[e, pl.ds(16 * k, 16)]
                    acc = acc + jnp.maximum(u + p, 0.0) * w[k]
                sv = jnp.where(lanes == i, jnp.sum(acc), sv)
            os_v[pl.ds(g * 16, 16)] = 5.0 / (1.0 + jnp.exp(-(sv + bias)))
            return carry2

        lax.fori_loop(0, CH // 16, strip, 0)

    g_start(c0, 0)
    npair = (c1 - c0 + 1) // 2

    def pair(t, carry):
        j0 = c0 + 2 * t
        j1 = j0 + 1
        g_wait(j0, 0)

        @pl.when(j1 < c1)
        def _():
            g_start(j1, 1)

        @pl.when(t >= 1)
        def _():
            o_wait(j0, 0)

        compute(j0, 0)
        o_start(j0, 0)

        @pl.when(j1 < c1)
        def _():
            g_wait(j1, 1)

            @pl.when(j1 + 1 < c1)
            def _():
                g_start(j1 + 1, 0)

            @pl.when(t >= 1)
            def _():
                o_wait(j1, 1)

            compute(j1, 1)
            o_start(j1, 1)

        return carry

    lax.fori_loop(0, npair, pair, 0)
    o_wait(0, 0)
    o_wait(0, 1)


def _row(i):
    return (i, 0)


def _full(i):
    return (0, 0)


def _dis_of(ca_r, cb_r):
    return lax.rsqrt(ca_r[:, :1] + cb_r[:, :1] + 1.0)


def _t1_body(uf_r, pf_r, uew_r, pew_r, uftW_r, uftb_r, pftW_r, pftb_r,
             c1W_r, ca_r, cb_r, hda_r, hdb_r, h1p_r):
    dis = _dis_of(ca_r, cb_r)
    ux = jnp.dot(uf_r[:], uftW_r[:], preferred_element_type=jnp.float32)
    ux = ux + uftb_r[:] + uew_r[:]
    h1u = jnp.dot(ux, c1W_r[:], preferred_element_type=jnp.float32)
    hd = h1u * dis
    hda_r[:] = hd[:, :DH]
    hdb_r[:] = hd[:, DH:]
    px = jnp.dot(pf_r[:], pftW_r[:], preferred_element_type=jnp.float32)
    px = px + pftb_r[:] + pew_r[:]
    h1p_r[:] = jnp.dot(px, c1W_r[:], preferred_element_type=jnp.float32)


def _t2_body(aa_r, ab_r, ca_r, cb_r, h1p_r, c1b_r, c2W_r,
             hda_r, hdb_r, h2p_r):
    dis = _dis_of(ca_r, cb_r)
    acc = jnp.concatenate([aa_r[:], ab_r[:]], axis=1)
    u1 = jnp.maximum(acc * dis + c1b_r[:], 0.0)
    h2u = jnp.dot(u1, c2W_r[:], preferred_element_type=jnp.float32)
    hd2 = h2u * dis
    hda_r[:] = hd2[:, :DH]
    hdb_r[:] = hd2[:, DH:]
    p1 = jnp.maximum(h1p_r[:] + c1b_r[:], 0.0)
    h2p_r[:] = jnp.dot(p1, c2W_r[:], preferred_element_type=jnp.float32)


def _t3_body(aa_r, ab_r, ca_r, cb_r, h2p_r, c2b_r, pWu_r, pWp_r, p1b_r,
             u2_r, p2_r):
    dis = _dis_of(ca_r, cb_r)
    u2 = jnp.concatenate([aa_r[:], ab_r[:]], axis=1) * dis + c2b_r[:]
    u2_r[:] = jnp.dot(u2, pWu_r[:], preferred_element_type=jnp.float32) + p1b_r[:]
    p2 = h2p_r[:] + c2b_r[:]
    p2_r[:] = jnp.dot(p2, pWp_r[:], preferred_element_type=jnp.float32)


def _tc_call(body, in_specs, out_specs, out_shape, args):
    return pl.pallas_call(
        body,
        grid=(NU // BM,),
        in_specs=in_specs,
        out_specs=out_specs,
        out_shape=out_shape,
    )(*args)


def kernel(edge_index, user_features, product_features, ue_w, pe_w, uft_W,
           uft_b, pft_W, pft_b, c1_W, c1_b, c2_W, c2_b, p1_W, p1_b, p2_W,
           p2_b):
    f32 = jnp.float32
    i32 = jnp.int32
    ei = edge_index.astype(i32)
    pad_e = jnp.zeros((ECHP - ECH, CH), i32)
    iu2d = jnp.concatenate([ei[0].reshape(ECH, CH), pad_e])
    ip2d = jnp.concatenate([ei[1].reshape(ECH, CH), pad_e])
    idx_flat = jnp.concatenate(
        [ei.reshape(TCH, CH), jnp.zeros((TCHP - TCH, CH), i32)])
    ones8 = jnp.ones((CH, 8), f32)
    zeros8 = jnp.zeros((HSL, 8), f32)

    cnt_a, cnt_b = _hist(idx_flat, ones8, zeros8)

    mrow = lambda shape: pl.BlockSpec(shape, _row)
    mfull = lambda shape: pl.BlockSpec(shape, _full)
    sd = jax.ShapeDtypeStruct

    hd1a, hd1b, h1p = _tc_call(
        _t1_body,
        [mrow((BM, 128)), mrow((BM, 128)), mrow((BM, D)), mrow((BM, D)),
         mfull((128, D)), mfull((1, D)), mfull((128, D)), mfull((1, D)),
         mfull((D, D)), mrow((BM, 8)), mrow((BM, 8))],
        [mrow((BM, DH)), mrow((BM, DH)), mrow((BM, D))],
        [sd((NUP, DH), f32), sd((NUP, DH), f32), sd((NU, D), f32)],
        (user_features, product_features, ue_w, pe_w, uft_W,
         uft_b.reshape(1, D), pft_W, pft_b.reshape(1, D), c1_W, cnt_a,
         cnt_b))

    acc1a, acc1b = _spmm(hd1a, hd1b, iu2d, ip2d)

    hd2a, hd2b, h2p = _tc_call(
        _t2_body,
        [mrow((BM, DH)), mrow((BM, DH)), mrow((BM, 8)), mrow((BM, 8)),
         mrow((BM, D)), mfull((1, D)), mfull((D, D))],
        [mrow((BM, DH)), mrow((BM, DH)), mrow((BM, D))],
        [sd((NUP, DH), f32), sd((NUP, DH), f32), sd((NU, D), f32)],
        (acc1a, acc1b, cnt_a, cnt_b, h1p, c1_b.reshape(1, D), c2_W))

    acc2a, acc2b = _spmm(hd2a, hd2b, iu2d, ip2d)

    u2, p2 = _tc_call(
        _t3_body,
        [mrow((BM, DH)), mrow((BM, DH)), mrow((BM, 8)), mrow((BM, 8)),
         mrow((BM, D)), mfull((1, D)), mfull((D, D)), mfull((D, D)),
         mfull((1, D))],
        [mrow((BM, D)), mrow((BM, D))],
        [sd((NU, D), f32), sd((NU, D), f32)],
        (acc2a, acc2b, cnt_a, cnt_b, h2p, c2_b.reshape(1, D), p1_W[:D],
         p1_W[D:], p1_b.reshape(1, D)))

    wb = jnp.concatenate([p2_W[:, 0], jnp.full((16,), p2_b[0], f32)])
    return _edge_score(u2, p2, iu2d, ip2d, wb)
